# Initial kernel scaffold; baseline (speedup 1.0000x reference)
import jax
import jax.numpy as jnp
from jax.experimental import pallas as pl


def kernel(pos, vel, acc, root_lin_vel, root_ang_vel, root_lin_acc, root_ang_acc, joint_centers, root_pos_history, root_euler_history, pre_bias, latent_bias, W_enc, W_dec):
    raise NotImplementedError("write your pallas kernel here")


# speedup vs baseline: 17.4415x; 17.4415x over previous
import functools

import jax
import jax.numpy as jnp
from jax.experimental import pallas as pl

B = 4096
D_IN = 210
N_LATENTS = 32768
K_SPARSITY = 100

BM0 = 256
BME, LNE = 256, 2048
BM2 = 128
BM3, LN3 = 256, 2048
N_BISECT = 26


def _norm_body(x_ref, pb_ref, xc_ref, mu_ref, std_ref):
    x = x_ref[...]
    mu = jnp.mean(x, axis=1, keepdims=True)
    std = jnp.sqrt(jnp.mean((x - mu) ** 2, axis=1, keepdims=True))
    xn = (x - mu) / (std + 1e-5)
    xc_ref[...] = xn - pb_ref[...]
    mu_ref[...] = mu
    std_ref[...] = std


def _enc_body(xc_ref, w_ref, b_ref, out_ref):
    out_ref[...] = (
        jnp.dot(xc_ref[...], w_ref[...], preferred_element_type=jnp.float32)
        + b_ref[...]
    )


def _select_body(pre_ref, th_ref):
    v = pre_ref[...]
    lo = jnp.min(v, axis=1, keepdims=True)
    hi = jnp.max(v, axis=1, keepdims=True)

    def step(_, carry):
        lo, hi = carry
        mid = 0.5 * (lo + hi)
        cnt = jnp.sum((v >= mid).astype(jnp.float32), axis=1, keepdims=True)
        ge = cnt >= K_SPARSITY
        return jnp.where(ge, mid, lo), jnp.where(ge, hi, mid)

    lo, hi = jax.lax.fori_loop(0, N_BISECT, step, (lo, hi))
    th_ref[...] = lo


def _finish_body(pre_ref, th_ref, w_ref, pb_ref, mu_ref, std_ref,
                 lat_ref, rec_ref, *, n_lat_blocks):
    l = pl.program_id(1)
    pre = pre_ref[...]
    lat = jnp.where(pre >= th_ref[...], pre, 0.0)
    lat_ref[...] = lat
    part = jnp.dot(lat, w_ref[...], preferred_element_type=jnp.float32)

    @pl.when(l == 0)
    def _():
        rec_ref[...] = part

    @pl.when(l > 0)
    def _():
        rec_ref[...] += part

    @pl.when(l == n_lat_blocks - 1)
    def _():
        rec_ref[...] = (rec_ref[...] + pb_ref[...]) * std_ref[...] + mu_ref[...]


def kernel(pos, vel, acc, root_lin_vel, root_ang_vel, root_lin_acc,
           root_ang_acc, joint_centers, root_pos_history, root_euler_history,
           pre_bias, latent_bias, W_enc, W_dec):
    x = jnp.concatenate([
        pos, vel, acc, root_lin_vel, root_ang_vel, root_lin_acc, root_ang_acc,
        joint_centers, root_pos_history, root_euler_history,
    ], axis=-1)
    b = x.shape[0]
    pb = pre_bias.reshape(1, D_IN)
    lb = latent_bias.reshape(1, N_LATENTS)

    xc, mu, std = pl.pallas_call(
        _norm_body,
        grid=(b // BM0,),
        in_specs=[
            pl.BlockSpec((BM0, D_IN), lambda i: (i, 0)),
            pl.BlockSpec((1, D_IN), lambda i: (0, 0)),
        ],
        out_specs=[
            pl.BlockSpec((BM0, D_IN), lambda i: (i, 0)),
            pl.BlockSpec((BM0, 1), lambda i: (i, 0)),
            pl.BlockSpec((BM0, 1), lambda i: (i, 0)),
        ],
        out_shape=[
            jax.ShapeDtypeStruct((b, D_IN), jnp.float32),
            jax.ShapeDtypeStruct((b, 1), jnp.float32),
            jax.ShapeDtypeStruct((b, 1), jnp.float32),
        ],
    )(x, pb)

    pre_act = pl.pallas_call(
        _enc_body,
        grid=(N_LATENTS // LNE, b // BME),
        in_specs=[
            pl.BlockSpec((BME, D_IN), lambda l, i: (i, 0)),
            pl.BlockSpec((D_IN, LNE), lambda l, i: (0, l)),
            pl.BlockSpec((1, LNE), lambda l, i: (0, l)),
        ],
        out_specs=pl.BlockSpec((BME, LNE), lambda l, i: (i, l)),
        out_shape=jax.ShapeDtypeStruct((b, N_LATENTS), jnp.float32),
    )(xc, W_enc, lb)

    thresh = pl.pallas_call(
        _select_body,
        grid=(b // BM2,),
        in_specs=[pl.BlockSpec((BM2, N_LATENTS), lambda i: (i, 0))],
        out_specs=pl.BlockSpec((BM2, 1), lambda i: (i, 0)),
        out_shape=jax.ShapeDtypeStruct((b, 1), jnp.float32),
    )(pre_act)

    n_lat_blocks = N_LATENTS // LN3
    latents, recons = pl.pallas_call(
        functools.partial(_finish_body, n_lat_blocks=n_lat_blocks),
        grid=(b // BM3, n_lat_blocks),
        in_specs=[
            pl.BlockSpec((BM3, LN3), lambda i, l: (i, l)),
            pl.BlockSpec((BM3, 1), lambda i, l: (i, 0)),
            pl.BlockSpec((LN3, D_IN), lambda i, l: (l, 0)),
            pl.BlockSpec((1, D_IN), lambda i, l: (0, 0)),
            pl.BlockSpec((BM3, 1), lambda i, l: (i, 0)),
            pl.BlockSpec((BM3, 1), lambda i, l: (i, 0)),
        ],
        out_specs=[
            pl.BlockSpec((BM3, LN3), lambda i, l: (i, l)),
            pl.BlockSpec((BM3, D_IN), lambda i, l: (i, 0)),
        ],
        out_shape=[
            jax.ShapeDtypeStruct((b, N_LATENTS), jnp.float32),
            jax.ShapeDtypeStruct((b, D_IN), jnp.float32),
        ],
    )(pre_act, thresh, W_dec, pb, mu, std)

    return pre_act, latents, recons
